# baseline (speedup 1.0000x reference)
import functools

import jax
import jax.numpy as jnp
from jax import lax
from jax.experimental import pallas as pl
from jax.experimental.pallas import tpu as pltpu
from jax.experimental.pallas import tpu_sc as plsc

N = 100000
EMB = 32
CH = 384
NSUB = 16
PER_TILE_ROWS = 3128
NLOC = NSUB * PER_TILE_ROWS
HALF = N // 2
DUMMY = 50000


def _cdiv(a, b):
    return (a + b - 1) // b


CHUNKS = _cdiv(N + 1600000, NSUB * CH)
EPAD = NSUB * CH * CHUNKS


def _edge_kernel(src_hbm, dst_hbm, eaw_hbm, xl_hbm, xr_hbm, we_hbm, att_hbm,
                 num_out, den_out,
                 num_sh, den_sh,
                 src_v, dst_v, eaw_v, xlr_v, xrr_v, denc_v,
                 idx2d_v, wev_v, attv_v, sem1, sem2):
    c = lax.axis_index("c")
    t = lax.axis_index("s")
    cbase = c * NLOC
    rbase = t * PER_TILE_ROWS

    zrow = jnp.zeros((16,), jnp.float32)
    for r in range(128):
        for k in range(2):
            xlr_v[r, pl.ds(k * 16, 16)] = zrow
    for i in range(CH // 16):
        denc_v[pl.ds(i * 16, 16)] = zrow
    for i in range(24):
        pltpu.sync_copy(xlr_v.at[pl.ds(0, 128)],
                        num_sh.at[pl.ds(rbase + i * 128, 128)])
    pltpu.sync_copy(xlr_v.at[pl.ds(0, 56)],
                    num_sh.at[pl.ds(rbase + 24 * 128, 56)])
    for i in range(8):
        pltpu.sync_copy(denc_v.at[pl.ds(0, CH)],
                        den_sh.at[pl.ds(rbase + i * CH, CH)])
    pltpu.sync_copy(denc_v.at[pl.ds(0, 56)],
                    den_sh.at[pl.ds(rbase + 8 * CH, 56)])
    plsc.subcore_barrier()

    pltpu.sync_copy(we_hbm, wev_v)
    pltpu.sync_copy(att_hbm, attv_v)
    we = [wev_v[0], wev_v[1]]
    at = [attv_v[0], attv_v[1]]

    lanes = lax.iota(jnp.int32, 16)
    chalf = c * HALF

    def chunk_body(ch, _):
        base = (t * CHUNKS + ch) * CH
        pltpu.sync_copy(src_hbm.at[pl.ds(base, CH)], src_v)
        pltpu.sync_copy(dst_hbm.at[pl.ds(base, CH)], dst_v)
        pltpu.sync_copy(eaw_hbm.at[pl.ds(base, CH)], eaw_v)
        cp1 = pltpu.async_copy(xl_hbm.at[src_v], xlr_v, sem1)
        cp2 = pltpu.async_copy(xr_hbm.at[dst_v], xrr_v, sem2)
        cp1.wait()
        cp2.wait()

        def group_body(g, _):
            ew = eaw_v[pl.ds(g * 16, 16)]
            evec = g * 16 + lanes
            alpha = jnp.zeros((16,), jnp.float32)
            for cc in range(EMB):
                cv = jnp.full((16,), cc, jnp.int32)
                xa = plsc.load_gather(xlr_v, [evec, cv])
                xb = plsc.load_gather(xrr_v, [evec, cv])
                k, l = divmod(cc, 16)
                tt = xa + xb + ew * we[k][l]
                tt = jnp.where(tt > 0, tt, 0.2 * tt)
                alpha = alpha + tt * at[k][l]
            wv = jnp.exp(alpha)
            denc_v[pl.ds(g * 16, 16)] = wv
            for cc in range(EMB):
                cv = jnp.full((16,), cc, jnp.int32)
                xa = plsc.load_gather(xlr_v, [evec, cv])
                plsc.store_scatter(xlr_v, [evec, cv], wv * xa)
            return _

        lax.fori_loop(0, CH // 16, group_body, None)

        for j in range(CH // 128):
            for k in range(8):
                d = dst_v[pl.ds(j * 128 + k * 16, 16)]
                loc = d - chalf
                ok = (loc >= 0) & (loc < HALF)
                idx2d_v[j, pl.ds(k * 16, 16)] = jnp.where(ok, loc, DUMMY)

        for j in range(CH // 128):
            pltpu.sync_copy(xlr_v.at[pl.ds(j * 128, 128)],
                            num_sh.at[idx2d_v.at[j]], add=True)
            pltpu.sync_copy(denc_v.at[pl.ds(j * 128, 128)],
                            den_sh.at[idx2d_v.at[j]], add=True)
        return _

    lax.fori_loop(0, CHUNKS, chunk_body, None)
    plsc.subcore_barrier()

    pltpu.sync_copy(num_sh.at[pl.ds(rbase, PER_TILE_ROWS)],
                    num_out.at[pl.ds(cbase + rbase, PER_TILE_ROWS)])
    pltpu.sync_copy(den_sh.at[pl.ds(rbase, PER_TILE_ROWS)],
                    den_out.at[pl.ds(cbase + rbase, PER_TILE_ROWS)])


def _sc_layer(src, dst, eaw, xl, xr, We, att):
    mesh = plsc.VectorSubcoreMesh(core_axis_name="c", subcore_axis_name="s")
    kfn = pl.kernel(
        _edge_kernel,
        mesh=mesh,
        compiler_params=pltpu.CompilerParams(needs_layout_passes=False,
                                             use_tc_tiling_on_sc=False),
        out_type=[
            jax.ShapeDtypeStruct((2 * NLOC, EMB), jnp.float32),
            jax.ShapeDtypeStruct((2 * NLOC,), jnp.float32),
        ],
        scratch_types=[
            pltpu.VMEM_SHARED((NLOC, EMB), jnp.float32),
            pltpu.VMEM_SHARED((NLOC,), jnp.float32),
            pltpu.VMEM((CH,), jnp.int32),
            pltpu.VMEM((CH,), jnp.int32),
            pltpu.VMEM((CH,), jnp.float32),
            pltpu.VMEM((CH, EMB), jnp.float32),
            pltpu.VMEM((CH, EMB), jnp.float32),
            pltpu.VMEM((CH,), jnp.float32),
            pltpu.VMEM((CH // 128, 128), jnp.int32),
            pltpu.VMEM((2, 16), jnp.float32),
            pltpu.VMEM((2, 16), jnp.float32),
            pltpu.SemaphoreType.DMA,
            pltpu.SemaphoreType.DMA,
        ],
    )
    return kfn(src, dst, eaw, xl, xr, We, att)


def _mean_kernel(ea_ref, out_ref):
    out_ref[0, 0] = jnp.sum(ea_ref[...]) / 1600000.0


def _proj_kernel(x_ref, wl_ref, wr_ref, xl_ref, xr_ref):
    x = x_ref[...]
    xl_ref[...] = jnp.dot(x, wl_ref[...], preferred_element_type=jnp.float32)
    xr_ref[...] = jnp.dot(x, wr_ref[...], preferred_element_type=jnp.float32)


def _proj2_kernel(num_ref, den_ref, b_ref, wl_ref, wr_ref, xl_ref, xr_ref):
    h = jax.nn.relu(num_ref[...] / den_ref[...] + b_ref[...])
    xl_ref[...] = jnp.dot(h, wl_ref[...], preferred_element_type=jnp.float32)
    xr_ref[...] = jnp.dot(h, wr_ref[...], preferred_element_type=jnp.float32)


def _dots_kernel(num_ref, den_ref, b_ref, msg_ref, wfc_ref, bfc_ref,
                 dots_ref, mx_ref, sm_ref, m_acc, s_acc):
    k = pl.program_id(0)

    @pl.when(k == 0)
    def _init():
        m_acc[...] = jnp.full_like(m_acc, -jnp.inf)
        s_acc[...] = jnp.zeros_like(s_acc)

    h = jax.nn.relu(num_ref[...] / den_ref[...] + b_ref[...])
    me = jnp.dot(msg_ref[...], wfc_ref[...],
                 preferred_element_type=jnp.float32) + bfc_ref[...]
    dots = jnp.dot(h, me.T, preferred_element_type=jnp.float32)
    dots_ref[...] = dots
    bm = jnp.max(dots, axis=0, keepdims=True)
    m_old = m_acc[...]
    m_new = jnp.maximum(m_old, bm)
    s_acc[...] = s_acc[...] * jnp.exp(m_old - m_new) + jnp.sum(
        jnp.exp(dots - m_new), axis=0, keepdims=True)
    m_acc[...] = m_new
    mx_ref[...] = m_new
    sm_ref[...] = s_acc[...]


def _norm_kernel(dots_ref, mx_ref, sm_ref, out_ref):
    out_ref[...] = jnp.exp(dots_ref[...] - mx_ref[...]) / sm_ref[...]


def _proj1(x, Wl, Wr):
    bn = 10000
    return pl.pallas_call(
        _proj_kernel,
        grid=(N // bn,),
        in_specs=[
            pl.BlockSpec((bn, x.shape[1]), lambda k: (k, 0)),
            pl.BlockSpec(Wl.shape, lambda k: (0, 0)),
            pl.BlockSpec(Wr.shape, lambda k: (0, 0)),
        ],
        out_specs=[
            pl.BlockSpec((bn, EMB), lambda k: (k, 0)),
            pl.BlockSpec((bn, EMB), lambda k: (k, 0)),
        ],
        out_shape=[
            jax.ShapeDtypeStruct((N, EMB), jnp.float32),
            jax.ShapeDtypeStruct((N, EMB), jnp.float32),
        ],
    )(x, Wl, Wr)


def _proj2(num, den, b, Wl, Wr):
    bn = 10000
    return pl.pallas_call(
        _proj2_kernel,
        grid=(N // bn,),
        in_specs=[
            pl.BlockSpec((bn, EMB), lambda k: (k, 0)),
            pl.BlockSpec((bn, 1), lambda k: (k, 0)),
            pl.BlockSpec((1, EMB), lambda k: (0, 0)),
            pl.BlockSpec(Wl.shape, lambda k: (0, 0)),
            pl.BlockSpec(Wr.shape, lambda k: (0, 0)),
        ],
        out_specs=[
            pl.BlockSpec((bn, EMB), lambda k: (k, 0)),
            pl.BlockSpec((bn, EMB), lambda k: (k, 0)),
        ],
        out_shape=[
            jax.ShapeDtypeStruct((N, EMB), jnp.float32),
            jax.ShapeDtypeStruct((N, EMB), jnp.float32),
        ],
    )(num, den, b, Wl, Wr)


def _mean_ea(edge_attr):
    ea = edge_attr.reshape(12500, 128)
    return pl.pallas_call(
        _mean_kernel,
        out_specs=pl.BlockSpec(memory_space=pltpu.SMEM),
        out_shape=jax.ShapeDtypeStruct((1, 1), jnp.float32),
    )(ea)


def _final_stage(num, den, b2, message, Wfc, bfc):
    b = message.shape[0]
    nblk = 10
    bn = N // nblk
    dots, mx, sm = pl.pallas_call(
        _dots_kernel,
        grid=(nblk,),
        in_specs=[
            pl.BlockSpec((bn, EMB), lambda k: (k, 0)),
            pl.BlockSpec((bn, 1), lambda k: (k, 0)),
            pl.BlockSpec((1, EMB), lambda k: (0, 0)),
            pl.BlockSpec(message.shape, lambda k: (0, 0)),
            pl.BlockSpec(Wfc.shape, lambda k: (0, 0)),
            pl.BlockSpec((1, EMB), lambda k: (0, 0)),
        ],
        out_specs=[
            pl.BlockSpec((bn, b), lambda k: (k, 0)),
            pl.BlockSpec((1, b), lambda k: (0, 0)),
            pl.BlockSpec((1, b), lambda k: (0, 0)),
        ],
        out_shape=[
            jax.ShapeDtypeStruct((N, b), jnp.float32),
            jax.ShapeDtypeStruct((1, b), jnp.float32),
            jax.ShapeDtypeStruct((1, b), jnp.float32),
        ],
        scratch_shapes=[
            pltpu.VMEM((1, b), jnp.float32),
            pltpu.VMEM((1, b), jnp.float32),
        ],
    )(num, den, b2, message, Wfc, bfc.reshape(1, -1))
    return pl.pallas_call(
        _norm_kernel,
        grid=(nblk,),
        in_specs=[
            pl.BlockSpec((bn, b), lambda k: (k, 0)),
            pl.BlockSpec((1, b), lambda k: (0, 0)),
            pl.BlockSpec((1, b), lambda k: (0, 0)),
        ],
        out_specs=pl.BlockSpec((bn, b), lambda k: (k, 0)),
        out_shape=jax.ShapeDtypeStruct((N, b), jnp.float32),
    )(dots, mx, sm)


def _slice_layer_out(nump, denp):
    num = jnp.concatenate([nump[:HALF], nump[NLOC:NLOC + HALF]], axis=0)
    den = jnp.concatenate([denp[:HALF], denp[NLOC:NLOC + HALF]])
    return num, den.reshape(N, 1)


def kernel(message, x, edge_index, edge_attr, Wl1, Wr1, We1, att1, b1,
           Wl2, Wr2, We2, att2, b2, Wfc, bfc):
    mean = _mean_ea(edge_attr)[0, 0]
    E = edge_index.shape[1]
    loop = jnp.arange(N, dtype=edge_index.dtype)
    pad = EPAD - (E + N)
    src = jnp.concatenate([edge_index[0], loop,
                           jnp.zeros((pad,), edge_index.dtype)])
    dst = jnp.concatenate([edge_index[1], loop,
                           jnp.full((pad,), N, edge_index.dtype)])
    eaw = jnp.concatenate([edge_attr[:, 0], jnp.full((N,), mean),
                           jnp.zeros((pad,), jnp.float32)])

    xl1, xr1 = _proj1(x, Wl1, Wr1)
    nump, denp = _sc_layer(src, dst, eaw, xl1, xr1,
                           We1.reshape(2, 16), att1.reshape(2, 16))
    num1, den1 = _slice_layer_out(nump, denp)

    xl2, xr2 = _proj2(num1, den1, b1.reshape(1, EMB), Wl2, Wr2)
    nump2, denp2 = _sc_layer(src, dst, eaw, xl2, xr2,
                             We2.reshape(2, 16), att2.reshape(2, 16))
    num2, den2 = _slice_layer_out(nump2, denp2)

    return _final_stage(num2, den2, b2.reshape(1, EMB), message, Wfc, bfc)


# speedup vs baseline: 2.1597x; 2.1597x over previous
import functools

import jax
import jax.numpy as jnp
from jax import lax
from jax.experimental import pallas as pl
from jax.experimental.pallas import tpu as pltpu
from jax.experimental.pallas import tpu_sc as plsc

N = 100000
EMB = 32
CH = 384
NSUB = 16
PER_TILE_ROWS = 3128
NLOC = NSUB * PER_TILE_ROWS
HALF = N // 2
DUMMY = 50000


def _cdiv(a, b):
    return (a + b - 1) // b


CHUNKS = _cdiv(N + 1600000, NSUB * CH)
EPAD = NSUB * CH * CHUNKS


def _edge_kernel(src_hbm, dst_hbm, eaw_hbm, xl_hbm, xr_hbm, we_hbm, att_hbm,
                 num_out, den_out,
                 num_sh, den_sh,
                 src_v, dst_v, eaw_v, xlr_v, xrr_v, denc_v,
                 idx2d_v, wev_v, attv_v, sem1, sem2):
    c = lax.axis_index("c")
    t = lax.axis_index("s")
    cbase = c * NLOC
    rbase = t * PER_TILE_ROWS

    zrow = jnp.zeros((16,), jnp.float32)
    for r in range(128):
        for k in range(2):
            xlr_v[r, pl.ds(k * 16, 16)] = zrow
    for i in range(CH // 16):
        denc_v[pl.ds(i * 16, 16)] = zrow
    for i in range(24):
        pltpu.sync_copy(xlr_v.at[pl.ds(0, 128)],
                        num_sh.at[pl.ds(rbase + i * 128, 128)])
    pltpu.sync_copy(xlr_v.at[pl.ds(0, 56)],
                    num_sh.at[pl.ds(rbase + 24 * 128, 56)])
    for i in range(8):
        pltpu.sync_copy(denc_v.at[pl.ds(0, CH)],
                        den_sh.at[pl.ds(rbase + i * CH, CH)])
    pltpu.sync_copy(denc_v.at[pl.ds(0, 56)],
                    den_sh.at[pl.ds(rbase + 8 * CH, 56)])
    plsc.subcore_barrier()

    pltpu.sync_copy(we_hbm, wev_v)
    pltpu.sync_copy(att_hbm, attv_v)
    we = [wev_v[0], wev_v[1]]
    at = [attv_v[0], attv_v[1]]

    lanes = lax.iota(jnp.int32, 16)
    chalf = c * HALF

    def chunk_body(ch, _):
        base = (t * CHUNKS + ch) * CH
        pltpu.sync_copy(src_hbm.at[pl.ds(base, CH)], src_v)
        pltpu.sync_copy(dst_hbm.at[pl.ds(base, CH)], dst_v)
        pltpu.sync_copy(eaw_hbm.at[pl.ds(base, CH)], eaw_v)
        cp1 = pltpu.async_copy(xl_hbm.at[src_v], xlr_v, sem1)
        cp2 = pltpu.async_copy(xr_hbm.at[dst_v], xrr_v, sem2)
        cp1.wait()
        cp2.wait()

        def group_body(g, _):
            ew = eaw_v[pl.ds(g * 16, 16)]
            wsel = jnp.zeros((16,), jnp.float32)
            for j in range(16):
                e = g * 16 + j
                x0 = xlr_v[e, pl.ds(0, 16)]
                x1 = xlr_v[e, pl.ds(16, 16)]
                t0 = x0 + xrr_v[e, pl.ds(0, 16)] + ew[j] * we[0]
                t1 = x1 + xrr_v[e, pl.ds(16, 16)] + ew[j] * we[1]
                t0 = jnp.where(t0 > 0, t0, 0.2 * t0)
                t1 = jnp.where(t1 > 0, t1, 0.2 * t1)
                alpha = jnp.sum(t0 * at[0] + t1 * at[1])
                wv = jnp.exp(jnp.full((16,), alpha))
                xlr_v[e, pl.ds(0, 16)] = wv * x0
                xlr_v[e, pl.ds(16, 16)] = wv * x1
                wsel = jnp.where(lanes == j, wv, wsel)
            denc_v[pl.ds(g * 16, 16)] = wsel
            return _

        lax.fori_loop(0, CH // 16, group_body, None)

        for j in range(CH // 128):
            for k in range(8):
                d = dst_v[pl.ds(j * 128 + k * 16, 16)]
                loc = d - chalf
                ok = (loc >= 0) & (loc < HALF)
                idx2d_v[j, pl.ds(k * 16, 16)] = jnp.where(ok, loc, DUMMY)

        for j in range(CH // 128):
            pltpu.sync_copy(xlr_v.at[pl.ds(j * 128, 128)],
                            num_sh.at[idx2d_v.at[j]], add=True)
            pltpu.sync_copy(denc_v.at[pl.ds(j * 128, 128)],
                            den_sh.at[idx2d_v.at[j]], add=True)
        return _

    lax.fori_loop(0, CHUNKS, chunk_body, None)
    plsc.subcore_barrier()

    pltpu.sync_copy(num_sh.at[pl.ds(rbase, PER_TILE_ROWS)],
                    num_out.at[pl.ds(cbase + rbase, PER_TILE_ROWS)])
    pltpu.sync_copy(den_sh.at[pl.ds(rbase, PER_TILE_ROWS)],
                    den_out.at[pl.ds(cbase + rbase, PER_TILE_ROWS)])


def _sc_layer(src, dst, eaw, xl, xr, We, att):
    mesh = plsc.VectorSubcoreMesh(core_axis_name="c", subcore_axis_name="s")
    kfn = pl.kernel(
        _edge_kernel,
        mesh=mesh,
        compiler_params=pltpu.CompilerParams(needs_layout_passes=False,
                                             use_tc_tiling_on_sc=False),
        out_type=[
            jax.ShapeDtypeStruct((2 * NLOC, EMB), jnp.float32),
            jax.ShapeDtypeStruct((2 * NLOC,), jnp.float32),
        ],
        scratch_types=[
            pltpu.VMEM_SHARED((NLOC, EMB), jnp.float32),
            pltpu.VMEM_SHARED((NLOC,), jnp.float32),
            pltpu.VMEM((CH,), jnp.int32),
            pltpu.VMEM((CH,), jnp.int32),
            pltpu.VMEM((CH,), jnp.float32),
            pltpu.VMEM((CH, EMB), jnp.float32),
            pltpu.VMEM((CH, EMB), jnp.float32),
            pltpu.VMEM((CH,), jnp.float32),
            pltpu.VMEM((CH // 128, 128), jnp.int32),
            pltpu.VMEM((2, 16), jnp.float32),
            pltpu.VMEM((2, 16), jnp.float32),
            pltpu.SemaphoreType.DMA,
            pltpu.SemaphoreType.DMA,
        ],
    )
    return kfn(src, dst, eaw, xl, xr, We, att)


def _mean_kernel(ea_ref, out_ref):
    out_ref[0, 0] = jnp.sum(ea_ref[...]) / 1600000.0


def _proj_kernel(x_ref, wl_ref, wr_ref, xl_ref, xr_ref):
    x = x_ref[...]
    xl_ref[...] = jnp.dot(x, wl_ref[...], preferred_element_type=jnp.float32)
    xr_ref[...] = jnp.dot(x, wr_ref[...], preferred_element_type=jnp.float32)


def _proj2_kernel(num_ref, den_ref, b_ref, wl_ref, wr_ref, xl_ref, xr_ref):
    h = jax.nn.relu(num_ref[...] / den_ref[...] + b_ref[...])
    xl_ref[...] = jnp.dot(h, wl_ref[...], preferred_element_type=jnp.float32)
    xr_ref[...] = jnp.dot(h, wr_ref[...], preferred_element_type=jnp.float32)


def _dots_kernel(num_ref, den_ref, b_ref, msg_ref, wfc_ref, bfc_ref,
                 dots_ref, mx_ref, sm_ref, m_acc, s_acc):
    k = pl.program_id(0)

    @pl.when(k == 0)
    def _init():
        m_acc[...] = jnp.full_like(m_acc, -jnp.inf)
        s_acc[...] = jnp.zeros_like(s_acc)

    h = jax.nn.relu(num_ref[...] / den_ref[...] + b_ref[...])
    me = jnp.dot(msg_ref[...], wfc_ref[...],
                 preferred_element_type=jnp.float32) + bfc_ref[...]
    dots = jnp.dot(h, me.T, preferred_element_type=jnp.float32)
    dots_ref[...] = dots
    bm = jnp.max(dots, axis=0, keepdims=True)
    m_old = m_acc[...]
    m_new = jnp.maximum(m_old, bm)
    s_acc[...] = s_acc[...] * jnp.exp(m_old - m_new) + jnp.sum(
        jnp.exp(dots - m_new), axis=0, keepdims=True)
    m_acc[...] = m_new
    mx_ref[...] = m_new
    sm_ref[...] = s_acc[...]


def _norm_kernel(dots_ref, mx_ref, sm_ref, out_ref):
    out_ref[...] = jnp.exp(dots_ref[...] - mx_ref[...]) / sm_ref[...]


def _proj1(x, Wl, Wr):
    bn = 10000
    return pl.pallas_call(
        _proj_kernel,
        grid=(N // bn,),
        in_specs=[
            pl.BlockSpec((bn, x.shape[1]), lambda k: (k, 0)),
            pl.BlockSpec(Wl.shape, lambda k: (0, 0)),
            pl.BlockSpec(Wr.shape, lambda k: (0, 0)),
        ],
        out_specs=[
            pl.BlockSpec((bn, EMB), lambda k: (k, 0)),
            pl.BlockSpec((bn, EMB), lambda k: (k, 0)),
        ],
        out_shape=[
            jax.ShapeDtypeStruct((N, EMB), jnp.float32),
            jax.ShapeDtypeStruct((N, EMB), jnp.float32),
        ],
    )(x, Wl, Wr)


def _proj2(num, den, b, Wl, Wr):
    bn = 10000
    return pl.pallas_call(
        _proj2_kernel,
        grid=(N // bn,),
        in_specs=[
            pl.BlockSpec((bn, EMB), lambda k: (k, 0)),
            pl.BlockSpec((bn, 1), lambda k: (k, 0)),
            pl.BlockSpec((1, EMB), lambda k: (0, 0)),
            pl.BlockSpec(Wl.shape, lambda k: (0, 0)),
            pl.BlockSpec(Wr.shape, lambda k: (0, 0)),
        ],
        out_specs=[
            pl.BlockSpec((bn, EMB), lambda k: (k, 0)),
            pl.BlockSpec((bn, EMB), lambda k: (k, 0)),
        ],
        out_shape=[
            jax.ShapeDtypeStruct((N, EMB), jnp.float32),
            jax.ShapeDtypeStruct((N, EMB), jnp.float32),
        ],
    )(num, den, b, Wl, Wr)


def _mean_ea(edge_attr):
    ea = edge_attr.reshape(12500, 128)
    return pl.pallas_call(
        _mean_kernel,
        out_specs=pl.BlockSpec(memory_space=pltpu.SMEM),
        out_shape=jax.ShapeDtypeStruct((1, 1), jnp.float32),
    )(ea)


def _final_stage(num, den, b2, message, Wfc, bfc):
    b = message.shape[0]
    nblk = 10
    bn = N // nblk
    dots, mx, sm = pl.pallas_call(
        _dots_kernel,
        grid=(nblk,),
        in_specs=[
            pl.BlockSpec((bn, EMB), lambda k: (k, 0)),
            pl.BlockSpec((bn, 1), lambda k: (k, 0)),
            pl.BlockSpec((1, EMB), lambda k: (0, 0)),
            pl.BlockSpec(message.shape, lambda k: (0, 0)),
            pl.BlockSpec(Wfc.shape, lambda k: (0, 0)),
            pl.BlockSpec((1, EMB), lambda k: (0, 0)),
        ],
        out_specs=[
            pl.BlockSpec((bn, b), lambda k: (k, 0)),
            pl.BlockSpec((1, b), lambda k: (0, 0)),
            pl.BlockSpec((1, b), lambda k: (0, 0)),
        ],
        out_shape=[
            jax.ShapeDtypeStruct((N, b), jnp.float32),
            jax.ShapeDtypeStruct((1, b), jnp.float32),
            jax.ShapeDtypeStruct((1, b), jnp.float32),
        ],
        scratch_shapes=[
            pltpu.VMEM((1, b), jnp.float32),
            pltpu.VMEM((1, b), jnp.float32),
        ],
    )(num, den, b2, message, Wfc, bfc.reshape(1, -1))
    return pl.pallas_call(
        _norm_kernel,
        grid=(nblk,),
        in_specs=[
            pl.BlockSpec((bn, b), lambda k: (k, 0)),
            pl.BlockSpec((1, b), lambda k: (0, 0)),
            pl.BlockSpec((1, b), lambda k: (0, 0)),
        ],
        out_specs=pl.BlockSpec((bn, b), lambda k: (k, 0)),
        out_shape=jax.ShapeDtypeStruct((N, b), jnp.float32),
    )(dots, mx, sm)


def _slice_layer_out(nump, denp):
    num = jnp.concatenate([nump[:HALF], nump[NLOC:NLOC + HALF]], axis=0)
    den = jnp.concatenate([denp[:HALF], denp[NLOC:NLOC + HALF]])
    return num, den.reshape(N, 1)


def kernel(message, x, edge_index, edge_attr, Wl1, Wr1, We1, att1, b1,
           Wl2, Wr2, We2, att2, b2, Wfc, bfc):
    mean = _mean_ea(edge_attr)[0, 0]
    E = edge_index.shape[1]
    loop = jnp.arange(N, dtype=edge_index.dtype)
    pad = EPAD - (E + N)
    src = jnp.concatenate([edge_index[0], loop,
                           jnp.zeros((pad,), edge_index.dtype)])
    dst = jnp.concatenate([edge_index[1], loop,
                           jnp.full((pad,), N, edge_index.dtype)])
    eaw = jnp.concatenate([edge_attr[:, 0], jnp.full((N,), mean),
                           jnp.zeros((pad,), jnp.float32)])

    xl1, xr1 = _proj1(x, Wl1, Wr1)
    nump, denp = _sc_layer(src, dst, eaw, xl1, xr1,
                           We1.reshape(2, 16), att1.reshape(2, 16))
    num1, den1 = _slice_layer_out(nump, denp)

    xl2, xr2 = _proj2(num1, den1, b1.reshape(1, EMB), Wl2, Wr2)
    nump2, denp2 = _sc_layer(src, dst, eaw, xl2, xr2,
                             We2.reshape(2, 16), att2.reshape(2, 16))
    num2, den2 = _slice_layer_out(nump2, denp2)

    return _final_stage(num2, den2, b2.reshape(1, EMB), message, Wfc, bfc)


# speedup vs baseline: 2.2897x; 1.0602x over previous
import functools

import jax
import jax.numpy as jnp
from jax import lax
from jax.experimental import pallas as pl
from jax.experimental.pallas import tpu as pltpu
from jax.experimental.pallas import tpu_sc as plsc

N = 100000
EMB = 32
CH = 384
NSUB = 16
PER_TILE_ROWS = 3128
NLOC = NSUB * PER_TILE_ROWS
HALF = N // 2
DUMMY = 50000


def _cdiv(a, b):
    return (a + b - 1) // b


CHUNKS = _cdiv(N + 1600000, NSUB * CH)
EPAD = NSUB * CH * CHUNKS


def _edge_kernel(src_hbm, dst_hbm, eaw_hbm, xl_hbm, xr_hbm, we_hbm, att_hbm,
                 num_out, den_out,
                 num_sh, den_sh,
                 src_v, dst_v, eaw_v, xlr_v, xrr_v, denc_v,
                 idx2d_v, wev_v, attv_v, sem1, sem2, sem3):
    c = lax.axis_index("c")
    t = lax.axis_index("s")
    cbase = c * NLOC
    rbase = t * PER_TILE_ROWS

    zrow = jnp.zeros((16,), jnp.float32)
    for r in range(128):
        for k in range(2):
            xlr_v[r, pl.ds(k * 16, 16)] = zrow
    for i in range(CH // 16):
        denc_v[pl.ds(i * 16, 16)] = zrow
    for i in range(24):
        pltpu.sync_copy(xlr_v.at[pl.ds(0, 128)],
                        num_sh.at[pl.ds(rbase + i * 128, 128)])
    pltpu.sync_copy(xlr_v.at[pl.ds(0, 56)],
                    num_sh.at[pl.ds(rbase + 24 * 128, 56)])
    for i in range(8):
        pltpu.sync_copy(denc_v.at[pl.ds(0, CH)],
                        den_sh.at[pl.ds(rbase + i * CH, CH)])
    pltpu.sync_copy(denc_v.at[pl.ds(0, 56)],
                    den_sh.at[pl.ds(rbase + 8 * CH, 56)])
    plsc.subcore_barrier()

    pltpu.sync_copy(we_hbm, wev_v)
    pltpu.sync_copy(att_hbm, attv_v)
    we = [wev_v[0], wev_v[1]]
    at = [attv_v[0], attv_v[1]]

    lanes = lax.iota(jnp.int32, 16)
    chalf = c * HALF

    def chunk_body(ch, _):
        base = (t * CHUNKS + ch) * CH
        cpa = pltpu.async_copy(src_hbm.at[pl.ds(base, CH)], src_v, sem1)
        cpb = pltpu.async_copy(dst_hbm.at[pl.ds(base, CH)], dst_v, sem2)
        cpc = pltpu.async_copy(eaw_hbm.at[pl.ds(base, CH)], eaw_v, sem3)
        cpa.wait()
        cpb.wait()
        cp1 = pltpu.async_copy(xl_hbm.at[src_v], xlr_v, sem1)
        cp2 = pltpu.async_copy(xr_hbm.at[dst_v], xrr_v, sem2)
        cpc.wait()
        cp1.wait()
        cp2.wait()

        def group_body(g, _):
            ew = eaw_v[pl.ds(g * 16, 16)]
            wsel = jnp.zeros((16,), jnp.float32)
            for j in range(16):
                e = g * 16 + j
                x0 = xlr_v[e, pl.ds(0, 16)]
                x1 = xlr_v[e, pl.ds(16, 16)]
                t0 = x0 + xrr_v[e, pl.ds(0, 16)] + ew[j] * we[0]
                t1 = x1 + xrr_v[e, pl.ds(16, 16)] + ew[j] * we[1]
                t0 = jnp.where(t0 > 0, t0, 0.2 * t0)
                t1 = jnp.where(t1 > 0, t1, 0.2 * t1)
                alpha = jnp.sum(t0 * at[0] + t1 * at[1])
                wv = jnp.exp(jnp.full((16,), alpha))
                xlr_v[e, pl.ds(0, 16)] = wv * x0
                xlr_v[e, pl.ds(16, 16)] = wv * x1
                wsel = jnp.where(lanes == j, wv, wsel)
            denc_v[pl.ds(g * 16, 16)] = wsel
            return _

        lax.fori_loop(0, CH // 16, group_body, None)

        for j in range(CH // 128):
            for k in range(8):
                d = dst_v[pl.ds(j * 128 + k * 16, 16)]
                loc = d - chalf
                ok = (loc >= 0) & (loc < HALF)
                idx2d_v[j, pl.ds(k * 16, 16)] = jnp.where(ok, loc, DUMMY)

        for j in range(CH // 128):
            pltpu.sync_copy(xlr_v.at[pl.ds(j * 128, 128)],
                            num_sh.at[idx2d_v.at[j]], add=True)
            pltpu.sync_copy(denc_v.at[pl.ds(j * 128, 128)],
                            den_sh.at[idx2d_v.at[j]], add=True)
        return _

    lax.fori_loop(0, CHUNKS, chunk_body, None)
    plsc.subcore_barrier()

    pltpu.sync_copy(num_sh.at[pl.ds(rbase, PER_TILE_ROWS)],
                    num_out.at[pl.ds(cbase + rbase, PER_TILE_ROWS)])
    pltpu.sync_copy(den_sh.at[pl.ds(rbase, PER_TILE_ROWS)],
                    den_out.at[pl.ds(cbase + rbase, PER_TILE_ROWS)])


def _sc_layer(src, dst, eaw, xl, xr, We, att):
    mesh = plsc.VectorSubcoreMesh(core_axis_name="c", subcore_axis_name="s")
    kfn = pl.kernel(
        _edge_kernel,
        mesh=mesh,
        compiler_params=pltpu.CompilerParams(needs_layout_passes=False,
                                             use_tc_tiling_on_sc=False),
        out_type=[
            jax.ShapeDtypeStruct((2 * NLOC, EMB), jnp.float32),
            jax.ShapeDtypeStruct((2 * NLOC,), jnp.float32),
        ],
        scratch_types=[
            pltpu.VMEM_SHARED((NLOC, EMB), jnp.float32),
            pltpu.VMEM_SHARED((NLOC,), jnp.float32),
            pltpu.VMEM((CH,), jnp.int32),
            pltpu.VMEM((CH,), jnp.int32),
            pltpu.VMEM((CH,), jnp.float32),
            pltpu.VMEM((CH, EMB), jnp.float32),
            pltpu.VMEM((CH, EMB), jnp.float32),
            pltpu.VMEM((CH,), jnp.float32),
            pltpu.VMEM((CH // 128, 128), jnp.int32),
            pltpu.VMEM((2, 16), jnp.float32),
            pltpu.VMEM((2, 16), jnp.float32),
            pltpu.SemaphoreType.DMA,
            pltpu.SemaphoreType.DMA,
            pltpu.SemaphoreType.DMA,
        ],
    )
    return kfn(src, dst, eaw, xl, xr, We, att)


def _mean_kernel(ea_ref, out_ref):
    out_ref[0, 0] = jnp.sum(ea_ref[...]) / 1600000.0


def _proj_kernel(x_ref, wl_ref, wr_ref, xl_ref, xr_ref):
    x = x_ref[...]
    xl_ref[...] = jnp.dot(x, wl_ref[...], preferred_element_type=jnp.float32)
    xr_ref[...] = jnp.dot(x, wr_ref[...], preferred_element_type=jnp.float32)


def _proj2_kernel(num_ref, den_ref, b_ref, wl_ref, wr_ref, xl_ref, xr_ref):
    h = jax.nn.relu(num_ref[...] / den_ref[...] + b_ref[...])
    xl_ref[...] = jnp.dot(h, wl_ref[...], preferred_element_type=jnp.float32)
    xr_ref[...] = jnp.dot(h, wr_ref[...], preferred_element_type=jnp.float32)


def _dots_kernel(num_ref, den_ref, b_ref, msg_ref, wfc_ref, bfc_ref,
                 dots_ref, mx_ref, sm_ref, m_acc, s_acc):
    k = pl.program_id(0)

    @pl.when(k == 0)
    def _init():
        m_acc[...] = jnp.full_like(m_acc, -jnp.inf)
        s_acc[...] = jnp.zeros_like(s_acc)

    h = jax.nn.relu(num_ref[...] / den_ref[...] + b_ref[...])
    me = jnp.dot(msg_ref[...], wfc_ref[...],
                 preferred_element_type=jnp.float32) + bfc_ref[...]
    dots = jnp.dot(h, me.T, preferred_element_type=jnp.float32)
    dots_ref[...] = dots
    bm = jnp.max(dots, axis=0, keepdims=True)
    m_old = m_acc[...]
    m_new = jnp.maximum(m_old, bm)
    s_acc[...] = s_acc[...] * jnp.exp(m_old - m_new) + jnp.sum(
        jnp.exp(dots - m_new), axis=0, keepdims=True)
    m_acc[...] = m_new
    mx_ref[...] = m_new
    sm_ref[...] = s_acc[...]


def _norm_kernel(dots_ref, mx_ref, sm_ref, out_ref):
    out_ref[...] = jnp.exp(dots_ref[...] - mx_ref[...]) / sm_ref[...]


def _proj1(x, Wl, Wr):
    bn = 10000
    return pl.pallas_call(
        _proj_kernel,
        grid=(N // bn,),
        in_specs=[
            pl.BlockSpec((bn, x.shape[1]), lambda k: (k, 0)),
            pl.BlockSpec(Wl.shape, lambda k: (0, 0)),
            pl.BlockSpec(Wr.shape, lambda k: (0, 0)),
        ],
        out_specs=[
            pl.BlockSpec((bn, EMB), lambda k: (k, 0)),
            pl.BlockSpec((bn, EMB), lambda k: (k, 0)),
        ],
        out_shape=[
            jax.ShapeDtypeStruct((N, EMB), jnp.float32),
            jax.ShapeDtypeStruct((N, EMB), jnp.float32),
        ],
    )(x, Wl, Wr)


def _proj2(num, den, b, Wl, Wr):
    bn = 10000
    return pl.pallas_call(
        _proj2_kernel,
        grid=(N // bn,),
        in_specs=[
            pl.BlockSpec((bn, EMB), lambda k: (k, 0)),
            pl.BlockSpec((bn, 1), lambda k: (k, 0)),
            pl.BlockSpec((1, EMB), lambda k: (0, 0)),
            pl.BlockSpec(Wl.shape, lambda k: (0, 0)),
            pl.BlockSpec(Wr.shape, lambda k: (0, 0)),
        ],
        out_specs=[
            pl.BlockSpec((bn, EMB), lambda k: (k, 0)),
            pl.BlockSpec((bn, EMB), lambda k: (k, 0)),
        ],
        out_shape=[
            jax.ShapeDtypeStruct((N, EMB), jnp.float32),
            jax.ShapeDtypeStruct((N, EMB), jnp.float32),
        ],
    )(num, den, b, Wl, Wr)


def _mean_ea(edge_attr):
    ea = edge_attr.reshape(12500, 128)
    return pl.pallas_call(
        _mean_kernel,
        out_specs=pl.BlockSpec(memory_space=pltpu.SMEM),
        out_shape=jax.ShapeDtypeStruct((1, 1), jnp.float32),
    )(ea)


def _final_stage(num, den, b2, message, Wfc, bfc):
    b = message.shape[0]
    nblk = 10
    bn = N // nblk
    dots, mx, sm = pl.pallas_call(
        _dots_kernel,
        grid=(nblk,),
        in_specs=[
            pl.BlockSpec((bn, EMB), lambda k: (k, 0)),
            pl.BlockSpec((bn, 1), lambda k: (k, 0)),
            pl.BlockSpec((1, EMB), lambda k: (0, 0)),
            pl.BlockSpec(message.shape, lambda k: (0, 0)),
            pl.BlockSpec(Wfc.shape, lambda k: (0, 0)),
            pl.BlockSpec((1, EMB), lambda k: (0, 0)),
        ],
        out_specs=[
            pl.BlockSpec((bn, b), lambda k: (k, 0)),
            pl.BlockSpec((1, b), lambda k: (0, 0)),
            pl.BlockSpec((1, b), lambda k: (0, 0)),
        ],
        out_shape=[
            jax.ShapeDtypeStruct((N, b), jnp.float32),
            jax.ShapeDtypeStruct((1, b), jnp.float32),
            jax.ShapeDtypeStruct((1, b), jnp.float32),
        ],
        scratch_shapes=[
            pltpu.VMEM((1, b), jnp.float32),
            pltpu.VMEM((1, b), jnp.float32),
        ],
    )(num, den, b2, message, Wfc, bfc.reshape(1, -1))
    return pl.pallas_call(
        _norm_kernel,
        grid=(nblk,),
        in_specs=[
            pl.BlockSpec((bn, b), lambda k: (k, 0)),
            pl.BlockSpec((1, b), lambda k: (0, 0)),
            pl.BlockSpec((1, b), lambda k: (0, 0)),
        ],
        out_specs=pl.BlockSpec((bn, b), lambda k: (k, 0)),
        out_shape=jax.ShapeDtypeStruct((N, b), jnp.float32),
    )(dots, mx, sm)


def _slice_layer_out(nump, denp):
    num = jnp.concatenate([nump[:HALF], nump[NLOC:NLOC + HALF]], axis=0)
    den = jnp.concatenate([denp[:HALF], denp[NLOC:NLOC + HALF]])
    return num, den.reshape(N, 1)


def kernel(message, x, edge_index, edge_attr, Wl1, Wr1, We1, att1, b1,
           Wl2, Wr2, We2, att2, b2, Wfc, bfc):
    mean = _mean_ea(edge_attr)[0, 0]
    E = edge_index.shape[1]
    loop = jnp.arange(N, dtype=edge_index.dtype)
    pad = EPAD - (E + N)
    src = jnp.concatenate([edge_index[0], loop,
                           jnp.zeros((pad,), edge_index.dtype)])
    dst = jnp.concatenate([edge_index[1], loop,
                           jnp.full((pad,), N, edge_index.dtype)])
    eaw = jnp.concatenate([edge_attr[:, 0], jnp.full((N,), mean),
                           jnp.zeros((pad,), jnp.float32)])

    xl1, xr1 = _proj1(x, Wl1, Wr1)
    nump, denp = _sc_layer(src, dst, eaw, xl1, xr1,
                           We1.reshape(2, 16), att1.reshape(2, 16))
    num1, den1 = _slice_layer_out(nump, denp)

    xl2, xr2 = _proj2(num1, den1, b1.reshape(1, EMB), Wl2, Wr2)
    nump2, denp2 = _sc_layer(src, dst, eaw, xl2, xr2,
                             We2.reshape(2, 16), att2.reshape(2, 16))
    num2, den2 = _slice_layer_out(nump2, denp2)

    return _final_stage(num2, den2, b2.reshape(1, EMB), message, Wfc, bfc)
